# Initial kernel scaffold; baseline (speedup 1.0000x reference)
import jax
import jax.numpy as jnp
from jax.experimental import pallas as pl


def kernel(node_feats, edge_index, edge_feats, glob_feats, batch, node_gamma, node_beta, edge_gamma, edge_beta, glob_gamma, glob_beta, eW1, eb1, eW2, eb2, nW1, nb1, nW2, nb2, gW1, gb1, gW2, gb2):
    raise NotImplementedError("write your pallas kernel here")


# speedup vs baseline: 2.1102x; 2.1102x over previous
import functools

import jax
import jax.numpy as jnp
from jax import lax
from jax.experimental import pallas as pl
from jax.experimental.pallas import tpu as pltpu
from jax.experimental.pallas import tpu_sc as plsc

_F32 = jnp.float32


def _onehot(batch_col, nseg):
    seg = lax.broadcasted_iota(jnp.int32, (1, nseg), 1)
    return (batch_col == seg).astype(_F32)


def _bn_norm(v, g, b):
    m = jnp.mean(v, axis=0, keepdims=True)
    var = jnp.mean(jnp.square(v - m), axis=0, keepdims=True)
    return (v - m) * lax.rsqrt(var + 1e-5) * g + b


def _bn_xu_body(x_ref, u_ref, xg_ref, xb_ref, ug_ref, ub_ref, xo_ref,
                uo_ref):
    xo_ref[...] = _bn_norm(x_ref[...], xg_ref[...], xb_ref[...])
    uo_ref[...] = _bn_norm(u_ref[...], ug_ref[...], ub_ref[...])


def _bn_xu_call(x, u, xg, xb, ug, ub):
    return pl.pallas_call(
        _bn_xu_body,
        out_shape=(
            jax.ShapeDtypeStruct(x.shape, _F32),
            jax.ShapeDtypeStruct(u.shape, _F32),
        ),
    )(x, u, xg, xb, ug, ub)


def _estats_body(e_ref, sum_ref, sq_ref):
    i = pl.program_id(0)

    @pl.when(i == 0)
    def _():
        sum_ref[...] = jnp.zeros_like(sum_ref)
        sq_ref[...] = jnp.zeros_like(sq_ref)

    v = e_ref[...]
    sum_ref[...] += jnp.sum(v, axis=0, keepdims=True)
    sq_ref[...] += jnp.sum(jnp.square(v), axis=0, keepdims=True)


def _enorm_body(e_ref, sum_ref, sq_ref, g_ref, b_ref, out_ref, *, ne):
    m = sum_ref[...] / ne
    var = sq_ref[...] / ne - jnp.square(m)
    out_ref[...] = ((e_ref[...] - m) * lax.rsqrt(var + 1e-5) * g_ref[...]
                    + b_ref[...])


def _bn_e_call(e_pad0, ne, eg, eb):
    ep, de = e_pad0.shape
    tile = 10000
    stats = pl.pallas_call(
        _estats_body,
        grid=(ne // tile,),
        in_specs=[pl.BlockSpec((tile, de), lambda i: (i, 0))],
        out_specs=(
            pl.BlockSpec((1, de), lambda i: (0, 0)),
            pl.BlockSpec((1, de), lambda i: (0, 0)),
        ),
        out_shape=(
            jax.ShapeDtypeStruct((1, de), _F32),
            jax.ShapeDtypeStruct((1, de), _F32),
        ),
    )(e_pad0)
    tile2 = 10240
    return pl.pallas_call(
        functools.partial(_enorm_body, ne=float(ne)),
        grid=(ep // tile2,),
        in_specs=[
            pl.BlockSpec((tile2, de), lambda i: (i, 0)),
            pl.BlockSpec((1, de), lambda i: (0, 0)),
            pl.BlockSpec((1, de), lambda i: (0, 0)),
            pl.BlockSpec((1, de), lambda i: (0, 0)),
            pl.BlockSpec((1, de), lambda i: (0, 0)),
        ],
        out_specs=pl.BlockSpec((tile2, de), lambda i: (i, 0)),
        out_shape=jax.ShapeDtypeStruct((ep, de), _F32),
    )(e_pad0, stats[0], stats[1], eg, eb)


def _tables_body(x_ref, bcol_ref, u_ref, ws_ref, wd_ref, wu_ref, b1_ref,
                 gs_ref, gd_ref):
    x = x_ref[...]
    uterm = jnp.dot(u_ref[...], wu_ref[...],
                    preferred_element_type=_F32) + b1_ref[...]
    oh = _onehot(bcol_ref[...], uterm.shape[0])
    gs_ref[...] = (jnp.dot(x, ws_ref[...], preferred_element_type=_F32)
                   + jnp.dot(oh, uterm, preferred_element_type=_F32))
    gd_ref[...] = jnp.dot(x, wd_ref[...], preferred_element_type=_F32)


def _tables_call(x, bcol, u, ws, wd, wu, b1):
    n, dn = x.shape
    eh = ws.shape[1]
    bsz = u.shape[0]
    tile = 1000
    grid = n // tile
    return pl.pallas_call(
        _tables_body,
        grid=(grid,),
        in_specs=[
            pl.BlockSpec((tile, dn), lambda i: (i, 0)),
            pl.BlockSpec((tile, 1), lambda i: (i, 0)),
            pl.BlockSpec((bsz, u.shape[1]), lambda i: (0, 0)),
            pl.BlockSpec((dn, eh), lambda i: (0, 0)),
            pl.BlockSpec((dn, eh), lambda i: (0, 0)),
            pl.BlockSpec((u.shape[1], eh), lambda i: (0, 0)),
            pl.BlockSpec((1, eh), lambda i: (0, 0)),
        ],
        out_specs=(
            pl.BlockSpec((tile, eh), lambda i: (i, 0)),
            pl.BlockSpec((tile, eh), lambda i: (i, 0)),
        ),
        out_shape=(
            jax.ShapeDtypeStruct((n, eh), _F32),
            jax.ShapeDtypeStruct((n, eh), _F32),
        ),
    )(x, bcol, u, ws, wd, wu, b1)


def _sc_gather_call(gsrc, gdst, src_pad, dst_pad):
    n, eh = gsrc.shape
    ep = src_pad.shape[0]
    info = plsc.get_sparse_core_info()
    nw = info.num_cores * info.num_subcores
    chunk = 128
    per_w = ep // nw
    n_chunks = per_w // chunk
    mesh = plsc.VectorSubcoreMesh(core_axis_name="c", subcore_axis_name="s")

    @functools.partial(
        pl.kernel,
        out_type=(
            jax.ShapeDtypeStruct((ep, eh), _F32),
            jax.ShapeDtypeStruct((ep, eh), _F32),
        ),
        mesh=mesh,
        scratch_types=[
            pltpu.VMEM((chunk,), jnp.int32),
            pltpu.VMEM((chunk,), jnp.int32),
            pltpu.VMEM((chunk, eh), _F32),
            pltpu.VMEM((chunk, eh), _F32),
            pltpu.SemaphoreType.DMA,
            pltpu.SemaphoreType.DMA,
        ],
    )
    def k(gs_h, gd_h, src_h, dst_h, o1_h, o2_h, si_v, di_v, r1_v, r2_v,
          sem1, sem2):
        wid = lax.axis_index("s") * info.num_cores + lax.axis_index("c")
        base = wid * per_w

        def body(c, carry):
            off = base + c * chunk
            pltpu.sync_copy(src_h.at[pl.ds(off, chunk)], si_v)
            pltpu.sync_copy(dst_h.at[pl.ds(off, chunk)], di_v)
            cp1 = pltpu.async_copy(gs_h.at[si_v], r1_v, sem1)
            cp2 = pltpu.async_copy(gd_h.at[di_v], r2_v, sem2)
            cp1.wait()
            cp2.wait()
            pltpu.sync_copy(r1_v, o1_h.at[pl.ds(off, chunk)])
            pltpu.sync_copy(r2_v, o2_h.at[pl.ds(off, chunk)])
            return carry

        lax.fori_loop(0, n_chunks, body, 0)

    return k(gsrc, gdst, src_pad, dst_pad)


def _edge_body(s1_ref, s2_ref, e_ref, we_ref, w2_ref, b2_ref, out_ref):
    h = s1_ref[...] + s2_ref[...] + jnp.dot(
        e_ref[...], we_ref[...], preferred_element_type=_F32)
    h = jnp.maximum(h, 0.0)
    out_ref[...] = jnp.dot(h, w2_ref[...],
                           preferred_element_type=_F32) + b2_ref[...]


def _edge_call(s1, s2, e, we, w2, b2):
    ep, eh = s1.shape
    de = e.shape[1]
    tile = 2048
    grid = ep // tile
    return pl.pallas_call(
        _edge_body,
        grid=(grid,),
        in_specs=[
            pl.BlockSpec((tile, eh), lambda i: (i, 0)),
            pl.BlockSpec((tile, eh), lambda i: (i, 0)),
            pl.BlockSpec((tile, de), lambda i: (i, 0)),
            pl.BlockSpec((de, eh), lambda i: (0, 0)),
            pl.BlockSpec((eh, de), lambda i: (0, 0)),
            pl.BlockSpec((1, de), lambda i: (0, 0)),
        ],
        out_specs=pl.BlockSpec((tile, de), lambda i: (i, 0)),
        out_shape=jax.ShapeDtypeStruct((ep, de), _F32),
    )(s1, s2, e, we, w2, b2)


def _lane_bcast(v, lane, j):
    return jnp.sum(jnp.where(lane == j, v, 0))


def _sc_scatter_call(rows, dst_pad, n_rows_padded):
    ep = dst_pad.shape[0]
    de = 16
    info = plsc.get_sparse_core_info()
    nc, ns = info.num_cores, info.num_subcores
    half = n_rows_padded // nc
    hr = half + 8
    chunk = 128
    per_tile = ep // ns
    n_chunks = per_tile // chunk
    flat_len = hr * de
    ones_mode = rows is None
    mesh = plsc.VectorSubcoreMesh(core_axis_name="c", subcore_axis_name="s")

    operands = (dst_pad,) if ones_mode else (rows, dst_pad)

    @functools.partial(
        pl.kernel,
        out_type=jax.ShapeDtypeStruct((ns * nc * flat_len,), _F32),
        mesh=mesh,
        scratch_types=[
            pltpu.VMEM((chunk,), jnp.int32),
            pltpu.VMEM((chunk, de), _F32),
            pltpu.VMEM((flat_len,), _F32),
        ],
        compiler_params=pltpu.CompilerParams(needs_layout_passes=False),
    )
    def k(*refs):
        if ones_mode:
            dst_h, out_h, di_v, rows_v, table = refs
            rows_h = None
        else:
            rows_h, dst_h, out_h, di_v, rows_v, table = refs
        cid = lax.axis_index("c")
        sid = lax.axis_index("s")
        base_e = sid * per_tile
        row_lo = cid * half
        lane = lax.broadcasted_iota(jnp.int32, (16,), 0)

        def zrow(i, carry):
            table[pl.ds(i * 16, 16)] = jnp.zeros((16,), _F32)
            return carry
        lax.fori_loop(0, flat_len // 16, zrow, 0)

        cval = (lane == 0).astype(_F32)

        def body(c, carry):
            off = base_e + c * chunk
            pltpu.sync_copy(dst_h.at[pl.ds(off, chunk)], di_v)
            if not ones_mode:
                pltpu.sync_copy(rows_h.at[pl.ds(off, chunk)], rows_v)

            def group(g, carry2):
                r16 = di_v[pl.ds(g * 16, 16)] - row_lo
                ok = (r16 >= 0) & (r16 < half)
                r16c = jnp.where(ok, r16, half)
                for j in range(16):
                    rb = _lane_bcast(r16c, lane, j)
                    fidx = rb * de + lane
                    if ones_mode:
                        val = cval
                    else:
                        val = rows_v[g * 16 + j, :]
                    plsc.addupdate_scatter(table, [fidx], val)
                return carry2

            lax.fori_loop(0, chunk // 16, group, 0)
            return carry

        lax.fori_loop(0, n_chunks, body, 0)
        wid = sid * nc + cid
        pltpu.sync_copy(table, out_h.at[pl.ds(wid * flat_len, flat_len)])

    return k(*operands).reshape(ns, nc, hr, de)


def _reduce_body(p_ref, o_ref):
    o_ref[...] = jnp.sum(p_ref[...], axis=0)


def _reduce_call(parts, n_rows_padded):
    ns, nc, hr, de = parts.shape
    half = n_rows_padded // nc
    tile = 1024
    out = pl.pallas_call(
        _reduce_body,
        grid=(nc, half // tile),
        in_specs=[pl.BlockSpec((ns, 1, tile, de), lambda h, i: (0, h, i, 0))],
        out_specs=pl.BlockSpec((1, tile, de), lambda h, i: (h, i, 0)),
        out_shape=jax.ShapeDtypeStruct((nc, half, de), _F32),
    )(parts)
    return out.reshape(n_rows_padded, de)


def _node_body(x_ref, p_ref, cnt_ref, bcol_ref, u_ref, w1x_ref, w1e_ref,
               w1u_ref, b1_ref, w2_ref, b2_ref, xo_ref, xs_ref):
    i = pl.program_id(0)
    inv = 1.0 / jnp.maximum(cnt_ref[:, 0:1], 1.0)
    eagg = p_ref[...] * inv
    uterm = jnp.dot(u_ref[...], w1u_ref[...], preferred_element_type=_F32)
    oh = _onehot(bcol_ref[...], uterm.shape[0])
    h = (jnp.dot(x_ref[...], w1x_ref[...], preferred_element_type=_F32)
         + jnp.dot(eagg, w1e_ref[...], preferred_element_type=_F32)
         + jnp.dot(oh, uterm, preferred_element_type=_F32)
         + b1_ref[...])
    h = jnp.maximum(h, 0.0)
    xn = jnp.dot(h, w2_ref[...], preferred_element_type=_F32) + b2_ref[...]
    xo_ref[...] = xn

    @pl.when(i == 0)
    def _():
        xs_ref[...] = jnp.zeros_like(xs_ref)

    xs_ref[...] += lax.dot_general(oh, xn, (((0,), (0,)), ((), ())),
                                   preferred_element_type=_F32)


def _node_call(x, p, cnt, bcol, u, w1x, w1e, w1u, b1, w2, b2):
    n, dn = x.shape
    nh = w1x.shape[1]
    de = w1e.shape[0]
    bsz = u.shape[0]
    tile = 1000
    grid = n // tile
    return pl.pallas_call(
        _node_body,
        grid=(grid,),
        in_specs=[
            pl.BlockSpec((tile, dn), lambda i: (i, 0)),
            pl.BlockSpec((tile, de), lambda i: (i, 0)),
            pl.BlockSpec((tile, de), lambda i: (i, 0)),
            pl.BlockSpec((tile, 1), lambda i: (i, 0)),
            pl.BlockSpec((bsz, u.shape[1]), lambda i: (0, 0)),
            pl.BlockSpec((dn, nh), lambda i: (0, 0)),
            pl.BlockSpec((de, nh), lambda i: (0, 0)),
            pl.BlockSpec((u.shape[1], nh), lambda i: (0, 0)),
            pl.BlockSpec((1, nh), lambda i: (0, 0)),
            pl.BlockSpec((nh, dn), lambda i: (0, 0)),
            pl.BlockSpec((1, dn), lambda i: (0, 0)),
        ],
        out_specs=(
            pl.BlockSpec((tile, dn), lambda i: (i, 0)),
            pl.BlockSpec((bsz, dn), lambda i: (0, 0)),
        ),
        out_shape=(
            jax.ShapeDtypeStruct((n, dn), _F32),
            jax.ShapeDtypeStruct((bsz, dn), _F32),
        ),
    )(x, p, cnt, bcol, u, w1x, w1e, w1u, b1, w2, b2)


def _glob_body(xs_ref, bcol_ref, u_ref, w1_ref, b1_ref, w2_ref, b2_ref,
               uo_ref):
    bsz = u_ref.shape[0]
    oh = _onehot(bcol_ref[...], bsz)
    bcnt = jnp.sum(oh, axis=0, keepdims=True)
    xm = xs_ref[...] / jnp.maximum(bcnt, 1.0).T
    gh = jnp.concatenate([xm, u_ref[...]], axis=1)
    h = jnp.maximum(
        jnp.dot(gh, w1_ref[...], preferred_element_type=_F32) + b1_ref[...],
        0.0)
    uo_ref[...] = jnp.dot(h, w2_ref[...],
                          preferred_element_type=_F32) + b2_ref[...]


def _glob_call(xs, bcol, u, w1, b1, w2, b2):
    bsz, du = u.shape
    return pl.pallas_call(
        _glob_body,
        out_shape=jax.ShapeDtypeStruct((bsz, du), _F32),
    )(xs, bcol, u, w1, b1, w2, b2)


def kernel(node_feats, edge_index, edge_feats, glob_feats, batch,
           node_gamma, node_beta, edge_gamma, edge_beta, glob_gamma,
           glob_beta, eW1, eb1, eW2, eb2, nW1, nb1, nW2, nb2, gW1, gb1,
           gW2, gb2):
    n, dn = node_feats.shape
    e_cnt, de = edge_feats.shape
    bsz, du = glob_feats.shape
    num_layers = eW1.shape[0]
    eh = eW1.shape[2]

    nw = 32
    chunk_rows = nw * 128
    ep = ((e_cnt + chunk_rows - 1) // chunk_rows) * chunk_rows
    npad = ((n + 1 + 255) // 256) * 256

    src = edge_index[0]
    dst = edge_index[1]
    pad_e = ep - e_cnt
    src_pad = jnp.concatenate([src, jnp.zeros((pad_e,), jnp.int32)])
    dst_pad_g = jnp.concatenate([dst, jnp.zeros((pad_e,), jnp.int32)])
    dst_pad_s = jnp.concatenate([dst, jnp.full((pad_e,), n, jnp.int32)])
    bcol = batch.reshape(n, 1)

    x, u = _bn_xu_call(
        node_feats, glob_feats,
        node_gamma.reshape(1, dn), node_beta.reshape(1, dn),
        glob_gamma.reshape(1, du), glob_beta.reshape(1, du))
    e_pad0 = jnp.pad(edge_feats, ((0, pad_e), (0, 0)))
    e = _bn_e_call(e_pad0, e_cnt, edge_gamma.reshape(1, de),
                   edge_beta.reshape(1, de))

    cnt = _reduce_call(_sc_scatter_call(None, dst_pad_s, npad), npad)

    for l in range(num_layers):
        ws = eW1[l, :dn]
        wd = eW1[l, dn:2 * dn]
        we = eW1[l, 2 * dn:2 * dn + de]
        wu = eW1[l, 2 * dn + de:]
        gsrc, gdst = _tables_call(x, bcol, u, ws, wd, wu,
                                  eb1[l].reshape(1, eh))
        s1, s2 = _sc_gather_call(gsrc, gdst, src_pad, dst_pad_g)
        e = _edge_call(s1, s2, e, we, eW2[l], eb2[l].reshape(1, de))
        p = _reduce_call(_sc_scatter_call(e, dst_pad_s, npad), npad)
        x, xs = _node_call(
            x, p, cnt, bcol, u,
            nW1[l, :dn], nW1[l, dn:dn + de], nW1[l, dn + de:],
            nb1[l].reshape(1, -1), nW2[l], nb2[l].reshape(1, dn))
        u = _glob_call(xs, bcol, u, gW1[l], gb1[l].reshape(1, -1),
                       gW2[l], gb2[l].reshape(1, du))

    return (x, e[:e_cnt], u)
